# Initial kernel scaffold; baseline (speedup 1.0000x reference)
import jax
import jax.numpy as jnp
from jax.experimental import pallas as pl


def kernel(node_ebd_ext, edge_ebd, h2, angle_ebd, nlist, nlist_mask, sw, angle_nlist, angle_nlist_mask, a_sw, W_self, b_self, W_sym, b_sym, W_ne, b_ne, W_ee, b_ee, r_n0, r_n1, r_n2, r_e0):
    raise NotImplementedError("write your pallas kernel here")


# speedup vs baseline: 1.7608x; 1.7608x over previous
import functools

import jax
import jax.numpy as jnp
from jax import lax
from jax.experimental import pallas as pl
from jax.experimental.pallas import tpu as pltpu
from jax.experimental.pallas import tpu_sc as plsc

AXIS = 4


def _sc_gather(table, idx, window):
    n_steps, _ = idx.shape
    d = table.shape[1]
    mesh = plsc.VectorSubcoreMesh(core_axis_name="core", subcore_axis_name="subcore")

    @functools.partial(
        pl.kernel,
        out_type=jax.ShapeDtypeStruct((n_steps * window, d), table.dtype),
        mesh=mesh,
    )
    def gather_kernel(x_hbm, i_hbm, o_hbm):
        def body(i_vmem, o_vmem):
            pltpu.sync_copy(x_hbm.at[i_vmem.at[0]], o_vmem)

        pltpu.emit_pipeline(
            body,
            grid=(n_steps,),
            in_specs=[pl.BlockSpec((1, window), index_map=lambda i: (i, 0))],
            out_specs=[pl.BlockSpec((window, d), index_map=lambda i: (i, 0))],
            core_axis_name=("core", "subcore"),
            dimension_semantics=(pltpu.PARALLEL,),
        )(i_hbm, o_hbm)

    return gather_kernel(table, idx)


def _tc_body(
    node_ref, nei_ref, edge_ref, cf_ref,
    w_self_ref, wc_node_ref, wc_nei_ref, wc_edge_ref, w_sym_ref,
    b_self_ref, b_c_ref, b_sym_ref,
    r_n0_ref, r_n1_ref, r_n2_ref, r_e0_ref,
    nout_ref, eout_ref,
):
    b = node_ref.shape[0]
    nnei = nei_ref.shape[0] // b
    e = b * nnei
    ndim = node_ref.shape[1]
    edim = edge_ref.shape[1]

    node = node_ref[...]
    nei = nei_ref[...]
    edge = edge_ref[...]
    cf = cf_ref[...]
    inv_nnei = 1.0 / nnei

    silu = jax.nn.silu

    node_self = silu(jnp.dot(node, w_self_ref[...],
                             preferred_element_type=jnp.float32) + b_self_ref[...])

    pre_n = jnp.dot(node, wc_node_ref[...], preferred_element_type=jnp.float32)
    pre = (
        jnp.dot(nei, wc_nei_ref[...], preferred_element_type=jnp.float32)
        + jnp.dot(edge, wc_edge_ref[...], preferred_element_type=jnp.float32)
        + b_c_ref[...]
    )
    nf = pre.shape[1]
    pre = pre + jnp.broadcast_to(
        pre_n[:, None, :], (b, nnei, nf)
    ).reshape(e, nf)
    eact = silu(pre)
    act_ne = eact[:, :ndim]
    edge_self = eact[:, ndim:ndim + edim]

    eout_ref[...] = edge + r_e0_ref[...] * edge_self

    swl = cf[:, 3:4]
    msg = jnp.sum((act_ne * swl).reshape(b, nnei, ndim), axis=1) * inv_nnei

    csw = cf * swl
    hgn = []
    hge = []
    for k in range(3):
        ck = csw[:, k:k + 1]
        hgn.append(jnp.sum((nei * ck).reshape(b, nnei, ndim), axis=1) * inv_nnei)
        hge.append(jnp.sum((edge * ck).reshape(b, nnei, edim), axis=1) * inv_nnei)

    sym_parts = []
    for a in range(AXIS):
        se = hge[0][:, a:a + 1] * hge[0]
        for k in range(1, 3):
            se = se + hge[k][:, a:a + 1] * hge[k]
        sym_parts.append(se)
    for a in range(AXIS):
        sn = hgn[0][:, a:a + 1] * hgn[0]
        for k in range(1, 3):
            sn = sn + hgn[k][:, a:a + 1] * hgn[k]
        sym_parts.append(sn)
    sym = jnp.concatenate(sym_parts, axis=-1)

    node_sym = silu(jnp.dot(sym, w_sym_ref[...],
                            preferred_element_type=jnp.float32) + b_sym_ref[...])

    nout_ref[...] = (
        node
        + r_n0_ref[...] * node_self
        + r_n1_ref[...] * node_sym
        + r_n2_ref[...] * msg
    )


def _tc_dense(node, nei_flat, edge_flat, cf, w_self, wc_node, wc_nei, wc_edge,
              w_sym_p, b_self, b_c, b_sym, r_n0, r_n1, r_n2, r_e0, block):
    nloc, ndim = node.shape
    e_tot, edim = edge_flat.shape
    nnei = e_tot // nloc
    nf = wc_node.shape[1]
    grid = (nloc // block,)
    eb = block * nnei

    full = lambda shape: pl.BlockSpec(shape, lambda i: (0, 0))
    out_shapes = (
        jax.ShapeDtypeStruct((nloc, ndim), jnp.float32),
        jax.ShapeDtypeStruct((e_tot, edim), jnp.float32),
    )
    return pl.pallas_call(
        _tc_body,
        grid=grid,
        in_specs=[
            pl.BlockSpec((block, ndim), lambda i: (i, 0)),
            pl.BlockSpec((eb, ndim), lambda i: (i, 0)),
            pl.BlockSpec((eb, edim), lambda i: (i, 0)),
            pl.BlockSpec((eb, 4), lambda i: (i, 0)),
            full((ndim, ndim)),
            full((ndim, nf)),
            full((ndim, nf)),
            full((edim, nf)),
            full((w_sym_p.shape[0], ndim)),
            full((1, ndim)),
            full((1, nf)),
            full((1, ndim)),
            full((1, ndim)),
            full((1, ndim)),
            full((1, ndim)),
            full((1, edim)),
        ],
        out_specs=[
            pl.BlockSpec((block, ndim), lambda i: (i, 0)),
            pl.BlockSpec((eb, edim), lambda i: (i, 0)),
        ],
        out_shape=out_shapes,
    )(node, nei_flat, edge_flat, cf, w_self, wc_node, wc_nei, wc_edge,
      w_sym_p, b_self, b_c, b_sym, r_n0, r_n1, r_n2, r_e0)


def _sym_perm(ndim, edim, axis):
    idx = []
    for a in range(axis):
        for d_ in range(edim):
            idx.append(d_ * axis + a)
    for a in range(axis):
        for d_ in range(ndim):
            idx.append(edim * axis + d_ * axis + a)
    return jnp.array(idx, dtype=jnp.int32)


def kernel(node_ebd_ext, edge_ebd, h2, angle_ebd, nlist, nlist_mask, sw,
           angle_nlist, angle_nlist_mask, a_sw, W_self, b_self, W_sym, b_sym,
           W_ne, b_ne, W_ee, b_ee, r_n0, r_n1, r_n2, r_e0):
    nb, nloc, nnei, edim = edge_ebd.shape
    ndim = node_ebd_ext.shape[-1]
    e_tot = nloc * nnei

    table = node_ebd_ext.reshape(-1, ndim)
    window = 400
    idx = nlist.reshape(e_tot // window, window).astype(jnp.int32)

    nei_flat = _sc_gather(table, idx, window=window)

    node = node_ebd_ext[0, :nloc, :]
    edge_flat = edge_ebd.reshape(e_tot, edim)
    sw_m = (sw * nlist_mask.astype(sw.dtype)).reshape(e_tot, 1)
    cf = jnp.concatenate([h2.reshape(e_tot, 3), sw_m], axis=-1)

    wc = jnp.concatenate([W_ne, W_ee], axis=1)
    wc_node = wc[:ndim]
    wc_nei = wc[ndim:2 * ndim]
    wc_edge = wc[2 * ndim:]
    b_c = jnp.concatenate([b_ne, b_ee]).reshape(1, -1)
    w_sym_p = W_sym[_sym_perm(ndim, edim, AXIS)]

    n_upd, e_upd = _tc_dense(
        node, nei_flat, edge_flat, cf, W_self, wc_node, wc_nei, wc_edge,
        w_sym_p, b_self.reshape(1, -1), b_c, b_sym.reshape(1, -1),
        r_n0.reshape(1, -1), r_n1.reshape(1, -1), r_n2.reshape(1, -1),
        r_e0.reshape(1, -1), block=200,
    )

    n_updated = n_upd.reshape(nb, nloc, ndim)
    e_updated = e_upd.reshape(nb, nloc, nnei, edim)
    return n_updated, e_updated, angle_ebd


# speedup vs baseline: 2.4199x; 1.3743x over previous
import functools

import jax
import jax.numpy as jnp
from jax.experimental import pallas as pl
from jax.experimental.pallas import tpu as pltpu
from jax.experimental.pallas import tpu_sc as plsc

AXIS = 4


def _sc_gather(table, idx, window):
    n_steps, _ = idx.shape
    d = table.shape[1]
    mesh = plsc.VectorSubcoreMesh(core_axis_name="core", subcore_axis_name="subcore")

    @functools.partial(
        pl.kernel,
        out_type=jax.ShapeDtypeStruct((n_steps * window, d), table.dtype),
        mesh=mesh,
    )
    def gather_kernel(x_hbm, i_hbm, o_hbm):
        def body(i_vmem, o_vmem):
            pltpu.sync_copy(x_hbm.at[i_vmem.at[0]], o_vmem)

        pltpu.emit_pipeline(
            body,
            grid=(n_steps,),
            in_specs=[pl.BlockSpec((1, window), index_map=lambda i: (i, 0))],
            out_specs=[pl.BlockSpec((window, d), index_map=lambda i: (i, 0))],
            core_axis_name=("core", "subcore"),
            dimension_semantics=(pltpu.PARALLEL,),
        )(i_hbm, o_hbm)

    return gather_kernel(table, idx)


def _tc_body(
    node_ref, nei_ref, edge_ref, cf_ref,
    w_self_ref, wc_node_ref, wc_nei_ref, wc_edge_ref, w_sym_ref,
    b_self_ref, b_c_ref, b_sym_ref,
    r_n0_ref, r_n1_ref, r_n2_ref, r_e0_ref,
    nout_ref, eout_ref,
):
    nnei, b, ndim = nei_ref.shape
    edim = edge_ref.shape[2]
    e = b * nnei
    inv_nnei = 1.0 / nnei

    node = node_ref[...]
    nei = nei_ref[...]
    edge = edge_ref[...]
    cf = cf_ref[...]

    silu = jax.nn.silu

    node_self = silu(jnp.dot(node, w_self_ref[...],
                             preferred_element_type=jnp.float32) + b_self_ref[...])

    pre_n = jnp.dot(node, wc_node_ref[...], preferred_element_type=jnp.float32)
    nf = pre_n.shape[1]
    pre = (
        jnp.dot(nei.reshape(e, ndim), wc_nei_ref[...],
                preferred_element_type=jnp.float32)
        + jnp.dot(edge.reshape(e, edim), wc_edge_ref[...],
                  preferred_element_type=jnp.float32)
    ).reshape(nnei, b, nf)
    eact = silu(pre + pre_n[None] + b_c_ref[...][None])

    eout_ref[...] = edge + r_e0_ref[...][None] * eact[:, :, ndim:ndim + edim]

    sw3 = cf[:, :, 3:4]
    csw = cf * sw3

    msg = jnp.sum(eact[:, :, :ndim] * sw3, axis=0) * inv_nnei
    hgn = [jnp.sum(nei * csw[:, :, k:k + 1], axis=0) * inv_nnei
           for k in range(3)]
    hge = [jnp.sum(edge * csw[:, :, k:k + 1], axis=0) * inv_nnei
           for k in range(3)]

    sym_parts = []
    for a in range(AXIS):
        se = hge[0][:, a:a + 1] * hge[0]
        for k in range(1, 3):
            se = se + hge[k][:, a:a + 1] * hge[k]
        sym_parts.append(se)
    for a in range(AXIS):
        sn = hgn[0][:, a:a + 1] * hgn[0]
        for k in range(1, 3):
            sn = sn + hgn[k][:, a:a + 1] * hgn[k]
        sym_parts.append(sn)
    sym = jnp.concatenate(sym_parts, axis=-1)

    node_sym = silu(jnp.dot(sym, w_sym_ref[...],
                            preferred_element_type=jnp.float32) + b_sym_ref[...])

    nout_ref[...] = (
        node
        + r_n0_ref[...] * node_self
        + r_n1_ref[...] * node_sym
        + r_n2_ref[...] * msg
    )


def _tc_dense(node, nei_t, edge_t, cf_t, w_self, wc_node, wc_nei, wc_edge,
              w_sym_p, b_self, b_c, b_sym, r_n0, r_n1, r_n2, r_e0, block):
    nloc, ndim = node.shape
    nnei, _, edim = edge_t.shape
    nf = wc_node.shape[1]
    grid = (nloc // block,)

    full = lambda shape: pl.BlockSpec(shape, lambda i: tuple(0 for _ in shape))
    out_shapes = (
        jax.ShapeDtypeStruct((nloc, ndim), jnp.float32),
        jax.ShapeDtypeStruct((nnei, nloc, edim), jnp.float32),
    )
    return pl.pallas_call(
        _tc_body,
        grid=grid,
        in_specs=[
            pl.BlockSpec((block, ndim), lambda i: (i, 0)),
            pl.BlockSpec((nnei, block, ndim), lambda i: (0, i, 0)),
            pl.BlockSpec((nnei, block, edim), lambda i: (0, i, 0)),
            pl.BlockSpec((nnei, block, 4), lambda i: (0, i, 0)),
            full((ndim, ndim)),
            full((ndim, nf)),
            full((ndim, nf)),
            full((edim, nf)),
            full((w_sym_p.shape[0], ndim)),
            full((1, ndim)),
            full((1, nf)),
            full((1, ndim)),
            full((1, ndim)),
            full((1, ndim)),
            full((1, ndim)),
            full((1, edim)),
        ],
        out_specs=[
            pl.BlockSpec((block, ndim), lambda i: (i, 0)),
            pl.BlockSpec((nnei, block, edim), lambda i: (0, i, 0)),
        ],
        out_shape=out_shapes,
    )(node, nei_t, edge_t, cf_t, w_self, wc_node, wc_nei, wc_edge,
      w_sym_p, b_self, b_c, b_sym, r_n0, r_n1, r_n2, r_e0)


def _sym_perm(ndim, edim, axis):
    idx = []
    for a in range(axis):
        for d_ in range(edim):
            idx.append(d_ * axis + a)
    for a in range(axis):
        for d_ in range(ndim):
            idx.append(edim * axis + d_ * axis + a)
    return jnp.array(idx, dtype=jnp.int32)


def kernel(node_ebd_ext, edge_ebd, h2, angle_ebd, nlist, nlist_mask, sw,
           angle_nlist, angle_nlist_mask, a_sw, W_self, b_self, W_sym, b_sym,
           W_ne, b_ne, W_ee, b_ee, r_n0, r_n1, r_n2, r_e0):
    nb, nloc, nnei, edim = edge_ebd.shape
    ndim = node_ebd_ext.shape[-1]
    e_tot = nloc * nnei

    table = node_ebd_ext.reshape(-1, ndim)
    window = 400
    idx = nlist[0].T.reshape(e_tot // window, window).astype(jnp.int32)

    nei_t = _sc_gather(table, idx, window=window).reshape(nnei, nloc, ndim)

    node = node_ebd_ext[0, :nloc, :]
    edge_t = jnp.transpose(edge_ebd[0], (1, 0, 2))
    sw_m = (sw * nlist_mask.astype(sw.dtype))[0].T[:, :, None]
    cf_t = jnp.concatenate([jnp.transpose(h2[0], (1, 0, 2)), sw_m], axis=-1)

    wc = jnp.concatenate([W_ne, W_ee], axis=1)
    wc_node = wc[:ndim]
    wc_nei = wc[ndim:2 * ndim]
    wc_edge = wc[2 * ndim:]
    b_c = jnp.concatenate([b_ne, b_ee]).reshape(1, -1)
    w_sym_p = W_sym[_sym_perm(ndim, edim, AXIS)]

    n_upd, e_upd_t = _tc_dense(
        node, nei_t, edge_t, cf_t, W_self, wc_node, wc_nei, wc_edge,
        w_sym_p, b_self.reshape(1, -1), b_c, b_sym.reshape(1, -1),
        r_n0.reshape(1, -1), r_n1.reshape(1, -1), r_n2.reshape(1, -1),
        r_e0.reshape(1, -1), block=200,
    )

    n_updated = n_upd.reshape(nb, nloc, ndim)
    e_updated = jnp.transpose(e_upd_t, (1, 0, 2)).reshape(nb, nloc, nnei, edim)
    return n_updated, e_updated, angle_ebd
